# baseline (speedup 1.0000x reference)
import functools

import jax
import jax.numpy as jnp
from jax import lax
from jax.experimental import pallas as pl
from jax.experimental.pallas import tpu as pltpu
from jax.experimental.pallas import tpu_sc as plsc

B = 4096
S = 8192
LANES = 16
NUM_CORES = 2
NUM_SUBCORES = 16
NUM_WORKERS = NUM_CORES * NUM_SUBCORES
ROWS_PER_WORKER = B // NUM_WORKERS
CHUNK = 4
NUM_CHUNKS = ROWS_PER_WORKER // CHUNK
UNROLL = 4
VREGS_PER_ROW = S // LANES

_mesh = plsc.VectorSubcoreMesh(core_axis_name="c", subcore_axis_name="s")


def _scan_row(buf, r):
    def body(jj, carry):
        base = pl.multiple_of(jj * (LANES * UNROLL), LANES * UNROLL)
        for u in range(UNROLL):
            v = buf[r, pl.ds(base + u * LANES, LANES)]
            buf[r, pl.ds(base + u * LANES, LANES)] = jnp.cumsum(v) + carry
            carry = carry + jnp.sum(v)
        return carry

    lax.fori_loop(0, VREGS_PER_ROW // UNROLL, body, jnp.float32(0.0),
                  unroll=1)


@functools.partial(
    pl.kernel,
    mesh=_mesh,
    out_type=jax.ShapeDtypeStruct((B, S), jnp.float32),
    scratch_types=[
        pltpu.VMEM((CHUNK, S), jnp.float32),
    ],
    compiler_params=pltpu.CompilerParams(needs_layout_passes=False),
)
def _cumsum_sc(x_hbm, out_hbm, buf):
    wid = lax.axis_index("s") * NUM_CORES + lax.axis_index("c")
    base_row = wid * ROWS_PER_WORKER

    def chunk_body(c, _):
        row0 = base_row + c * CHUNK
        pltpu.sync_copy(x_hbm.at[pl.ds(row0, CHUNK)], buf)
        for r in range(CHUNK):
            _scan_row(buf, r)
        pltpu.sync_copy(buf, out_hbm.at[pl.ds(row0, CHUNK)])
        return 0

    lax.fori_loop(0, NUM_CHUNKS, chunk_body, 0)


def kernel(x):
    return _cumsum_sc(x)


# speedup vs baseline: 1.2082x; 1.2082x over previous
import functools

import jax
import jax.numpy as jnp
from jax import lax
from jax.experimental import pallas as pl
from jax.experimental.pallas import tpu as pltpu
from jax.experimental.pallas import tpu_sc as plsc

B = 4096
S = 8192
LANES = 16
NUM_CORES = 2
NUM_SUBCORES = 16
NUM_WORKERS = NUM_CORES * NUM_SUBCORES
ROWS_PER_WORKER = B // NUM_WORKERS
CHUNK = 2
NBUF = 4
NUM_CHUNKS = ROWS_PER_WORKER // CHUNK
GROUPS = NUM_CHUNKS // NBUF
UNROLL = 4
VREGS_PER_ROW = S // LANES

_mesh = plsc.VectorSubcoreMesh(core_axis_name="c", subcore_axis_name="s")


def _scan_row(buf, r):
    def body(jj, carry):
        base = pl.multiple_of(jj * (LANES * UNROLL), LANES * UNROLL)
        for u in range(UNROLL):
            v = buf[r, pl.ds(base + u * LANES, LANES)]
            buf[r, pl.ds(base + u * LANES, LANES)] = jnp.cumsum(v) + carry
            carry = carry + jnp.sum(v)
        return carry

    lax.fori_loop(0, VREGS_PER_ROW // UNROLL, body, jnp.float32(0.0),
                  unroll=1)


@functools.partial(
    pl.kernel,
    mesh=_mesh,
    out_type=jax.ShapeDtypeStruct((B, S), jnp.float32),
    scratch_types=(
        [pltpu.VMEM((CHUNK, S), jnp.float32)] * NBUF
        + [pltpu.SemaphoreType.DMA] * (2 * NBUF)
    ),
    compiler_params=pltpu.CompilerParams(needs_layout_passes=False),
)
def _cumsum_sc(x_hbm, out_hbm, *scratch):
    bufs = scratch[:NBUF]
    lsems = scratch[NBUF:2 * NBUF]
    ssems = scratch[2 * NBUF:]

    wid = lax.axis_index("s") * NUM_CORES + lax.axis_index("c")
    base_row = wid * ROWS_PER_WORKER

    def start_load(c, b):
        row0 = base_row + c * CHUNK
        pltpu.make_async_copy(
            x_hbm.at[pl.ds(row0, CHUNK)], bufs[b], lsems[b]).start()

    def wait_load(b):
        pltpu.make_async_copy(
            x_hbm.at[pl.ds(base_row, CHUNK)], bufs[b], lsems[b]).wait()

    def start_store(c, b):
        row0 = base_row + c * CHUNK
        pltpu.make_async_copy(
            bufs[b], out_hbm.at[pl.ds(row0, CHUNK)], ssems[b]).start()

    def wait_store(b):
        pltpu.make_async_copy(
            bufs[b], out_hbm.at[pl.ds(base_row, CHUNK)], ssems[b]).wait()

    start_load(0, 0)

    def group_body(g, _):
        for u in range(NBUF):
            c = g * NBUF + u
            bn = (u + 1) % NBUF
            nc = c + 1

            @pl.when(nc < NUM_CHUNKS)
            def _prefetch():
                @pl.when(nc >= NBUF)
                def _drain():
                    wait_store(bn)
                start_load(nc, bn)

            wait_load(u)
            for r in range(CHUNK):
                _scan_row(bufs[u], r)
            start_store(c, u)
        return 0

    lax.fori_loop(0, GROUPS, group_body, 0)

    for b in range(NBUF):
        wait_store(b)


def kernel(x):
    return _cumsum_sc(x)
